# Initial kernel scaffold; baseline (speedup 1.0000x reference)
import jax
import jax.numpy as jnp
from jax.experimental import pallas as pl


def kernel(x_world, center, half, svf_L0, svf_L1):
    raise NotImplementedError("write your pallas kernel here")


# speedup vs baseline: 36.6099x; 36.6099x over previous
import functools

import jax
import jax.numpy as jnp
from jax.experimental import pallas as pl
from jax.experimental.pallas import tpu as pltpu
from jax.experimental.pallas import tpu_sc as plsc

_IDX_B = 2048
_CMB_B = 512
_GW = 128


def _build_table(svf):
    v = svf[0]
    _, D, H, W = v.shape
    vp = jnp.pad(v, ((0, 0), (0, 1), (0, 1), (0, 1)), mode="edge")
    cols = []
    for c in range(3):
        for k in range(8):
            dz, dy, dx = (k >> 2) & 1, (k >> 1) & 1, k & 1
            cols.append(vp[c, dz:dz + D, dy:dy + H, dx:dx + W].reshape(-1))
    t = jnp.stack(cols, axis=-1)
    return jnp.pad(t, ((0, 0), (0, 104)))


def _pos(xn, D):
    return jnp.clip((xn + 1.0) * (0.5 * (D - 1)), 0.0, float(D - 1))


def _norm(x, c, ih):
    return jnp.clip((x - c) * ih, -1.5, 1.5)


def _tc_indices(xT, cen, invh):
    G = xT.shape[1]
    B = _IDX_B

    def body(x_ref, c_ref, h_ref, i0_ref, i1_ref):
        xn = _norm(x_ref[...], c_ref[...], h_ref[...])

        def flat(D):
            i = jnp.floor(_pos(xn, D)).astype(jnp.int32)
            return (i[2:3, :] * D + i[1:2, :]) * D + i[0:1, :]

        i0_ref[...] = flat(64)
        i1_ref[...] = flat(128)

    return pl.pallas_call(
        body,
        grid=(G // B,),
        in_specs=[pl.BlockSpec((8, B), lambda w: (0, w)),
                  pl.BlockSpec((8, 1), lambda w: (0, 0)),
                  pl.BlockSpec((8, 1), lambda w: (0, 0))],
        out_specs=[pl.BlockSpec((1, B), lambda w: (0, w)),
                   pl.BlockSpec((1, B), lambda w: (0, w))],
        out_shape=[jax.ShapeDtypeStruct((1, G), jnp.int32),
                   jax.ShapeDtypeStruct((1, G), jnp.int32)],
    )(xT, cen, invh)


def _sc_gather(t0, i0, t1, i1):
    G = i0.shape[1]
    mesh = plsc.VectorSubcoreMesh(core_axis_name="core",
                                  subcore_axis_name="subcore")

    @functools.partial(
        pl.kernel,
        out_type=[jax.ShapeDtypeStruct((G, 128), jnp.float32),
                  jax.ShapeDtypeStruct((G, 128), jnp.float32)],
        mesh=mesh)
    def k(t0_hbm, i0_hbm, t1_hbm, i1_hbm, o0_hbm, o1_hbm):
        def body(i0_v, i1_v, o0_v, o1_v):
            pltpu.sync_copy(t0_hbm.at[i0_v.at[0]], o0_v)
            pltpu.sync_copy(t1_hbm.at[i1_v.at[0]], o1_v)

        pltpu.emit_pipeline(
            body,
            grid=(G // _GW,),
            in_specs=[pl.BlockSpec((1, _GW), lambda i: (0, i)),
                      pl.BlockSpec((1, _GW), lambda i: (0, i))],
            out_specs=[pl.BlockSpec((_GW, 128), lambda i: (i, 0)),
                       pl.BlockSpec((_GW, 128), lambda i: (i, 0))],
            core_axis_name=("core", "subcore"),
            dimension_semantics=(pltpu.PARALLEL,),
        )(i0_hbm, i1_hbm, o0_hbm, o1_hbm)

    return k(t0, i0, t1, i1)


def _tc_combine(xT, cen, invh, g0, g1):
    G = xT.shape[1]
    B = _CMB_B

    def body(x_ref, c_ref, h_ref, g0_ref, g1_ref, o_ref):
        xn = _norm(x_ref[...], c_ref[...], h_ref[...])
        acc = [jnp.zeros((1, B), jnp.float32) for _ in range(3)]
        for g_ref, D in ((g0_ref, 64), (g1_ref, 128)):
            pos = _pos(xn, D)
            fr = pos - jnp.floor(pos)
            wx, wy, wz = fr[0:1, :], fr[1:2, :], fr[2:3, :]
            t = jnp.transpose(g_ref[:, :32])
            az = (1.0 - wz, wz)
            ay = (1.0 - wy, wy)
            ax = (1.0 - wx, wx)
            for dz in range(2):
                for dy in range(2):
                    zy = az[dz] * ay[dy]
                    for dx in range(2):
                        w = zy * ax[dx]
                        k = dz * 4 + dy * 2 + dx
                        for c in range(3):
                            acc[c] = acc[c] + w * t[8 * c + k:8 * c + k + 1, :]
        o_ref[...] = jnp.concatenate(acc + [jnp.zeros((5, B), jnp.float32)],
                                     axis=0)

    return pl.pallas_call(
        body,
        grid=(G // B,),
        in_specs=[pl.BlockSpec((8, B), lambda w: (0, w)),
                  pl.BlockSpec((8, 1), lambda w: (0, 0)),
                  pl.BlockSpec((8, 1), lambda w: (0, 0)),
                  pl.BlockSpec((B, 128), lambda w: (w, 0)),
                  pl.BlockSpec((B, 128), lambda w: (w, 0))],
        out_specs=pl.BlockSpec((8, B), lambda w: (0, w)),
        out_shape=jax.ShapeDtypeStruct((8, G), jnp.float32),
    )(xT, cen, invh, g0, g1)


def kernel(x_world, center, half, svf_L0, svf_L1):
    G = x_world.shape[0]
    t0 = _build_table(svf_L0)
    t1 = _build_table(svf_L1)
    xT = jnp.pad(x_world.T, ((0, 5), (0, 0)))
    cen = jnp.pad(center, (0, 5)).reshape(8, 1)
    invh = jnp.pad(1.0 / (half + 1e-8), (0, 5)).reshape(8, 1)
    i0, i1 = _tc_indices(xT, cen, invh)
    g0, g1 = _sc_gather(t0, i0, t1, i1)
    outT = _tc_combine(xT, cen, invh, g0, g1)
    return outT[:3, :].T
